# baseline (speedup 1.0000x reference)
import functools

import jax
import jax.numpy as jnp
from jax import lax
from jax.experimental import pallas as pl
from jax.experimental.pallas import tpu as pltpu
from jax.experimental.pallas import tpu_sc as plsc

_B, _L, _D = 16384, 200, 16
_LD = _L * _D
_CB = 16


def _sc_body(x_hbm, pos_hbm, out_hbm, x_v, pos_v, out_v):
    info = plsc.get_sparse_core_info()
    nw = info.num_cores * info.num_subcores
    wid = lax.axis_index("s") * info.num_cores + lax.axis_index("c")
    rows = _B // nw
    pltpu.sync_copy(pos_hbm, pos_v)
    base0 = wid * rows

    def do_group(i, lane0, loff):
        x16 = x_v[i, pl.ds(loff, 16)]
        for j in range(lane0, 16):
            off = (loff + j) * _D
            out_v[i, pl.ds(off, _D)] = x16[j] + pos_v[pl.ds(off, _D)]

    def chunk(c, carry):
        base = base0 + c * _CB
        pltpu.sync_copy(x_hbm.at[pl.ds(base, _CB)], x_v)

        def per_group(g, carry2):
            loff = pl.multiple_of(g * 16, 16)
            for i in range(_CB):
                do_group(i, 0, loff)
            return carry2

        lax.fori_loop(0, _L // 16, per_group, 0)
        for i in range(_CB):
            do_group(i, 8, _L - 16)
        pltpu.sync_copy(out_v, out_hbm.at[pl.ds(base, _CB)])
        return carry

    lax.fori_loop(0, rows // _CB, chunk, 0)


def kernel(x, pos_table):
    B, L = x.shape
    D = pos_table.shape[-1]
    pos_flat = pos_table.reshape(L * D)
    k = functools.partial(
        pl.kernel,
        mesh=plsc.VectorSubcoreMesh(core_axis_name="c", subcore_axis_name="s"),
        out_type=jax.ShapeDtypeStruct((B, L * D), x.dtype),
        scratch_types=[
            pltpu.VMEM((_CB, L), jnp.float32),
            pltpu.VMEM((L * D,), jnp.float32),
            pltpu.VMEM((_CB, L * D), jnp.float32),
        ],
    )(_sc_body)
    y = k(x, pos_flat)
    return y.reshape(B, L, D)
